# baseline (speedup 1.0000x reference)
import jax
import jax.numpy as jnp
from jax.experimental import pallas as pl
from jax.experimental.pallas import tpu as pltpu

_B, _T_IN, _T_OUT, _N, _D, _H = 64, 32, 32, 64, 128, 64
_T_TOT = _T_IN + _T_OUT
_BB = 4
_R = _BB * _N
_NCHUNK = _B // _BB


def _dot(a, b):
    return jax.lax.dot(a, b, preferred_element_type=jnp.float32)


def _body(x_ref, Abd_ref,
          eWg0_ref, ebg0_ref, eWc0_ref, ebc0_ref,
          eWg1_ref, ebg1_ref, eWc1_ref, ebc1_ref,
          dWg0_ref, dbg0_ref, dWc0_ref, dbc0_ref,
          dWg1_ref, dbg1_ref, dWc1_ref, dbc1_ref,
          pW_ref, pb_ref,
          out_ref, h0_ref, h1_ref, cur_ref, convg_ref, convc_ref, rh_ref):
    t = pl.program_id(0)
    Abd = Abd_ref[...]

    @pl.when(t == 0)
    def _init():
        h0_ref[...] = jnp.zeros(h0_ref.shape, jnp.float32)
        h1_ref[...] = jnp.zeros(h1_ref.shape, jnp.float32)
        cur_ref[...] = jnp.zeros(cur_ref.shape, jnp.float32)

    def _cell(get_x, Dx, h_ref, Wg, bg, Wc, bc):
        W = Dx + _H
        for i in range(_NCHUNK):
            sl = slice(_R * i, _R * (i + 1))
            xh = jnp.concatenate([get_x(i), h_ref[sl, :]], axis=1)
            conv = _dot(Abd, xh)
            convg_ref[sl, :W] = conv
            convc_ref[sl, :Dx] = conv[:, :Dx]
        g = jax.nn.sigmoid(_dot(convg_ref[:, :W], Wg) + bg)
        r = g[:, :_H]
        u = g[:, _H:]
        rh_ref[...] = r * h_ref[...]
        for i in range(_NCHUNK):
            sl = slice(_R * i, _R * (i + 1))
            convc_ref[sl, Dx:W] = _dot(Abd, rh_ref[sl, :])
        c = jnp.tanh(_dot(convc_ref[:, :W], Wc) + bc)
        hn = u * h_ref[...] + (1.0 - u) * c
        h_ref[...] = hn
        return hn

    def _two_layers(get_x0, Wg0, bg0, Wc0, bc0, Wg1, bg1, Wc1, bc1):
        _cell(get_x0, _D, h0_ref, Wg0, bg0, Wc0, bc0)
        get_h0 = lambda i: h0_ref[slice(_R * i, _R * (i + 1)), :]
        return _cell(get_h0, _H, h1_ref, Wg1, bg1, Wc1, bc1)

    @pl.when(t < _T_IN)
    def _enc():
        get_x = lambda i: x_ref[0, _BB * i:_BB * (i + 1)].reshape(_R, _D)
        _two_layers(get_x,
                    eWg0_ref[...], ebg0_ref[...], eWc0_ref[...], ebc0_ref[...],
                    eWg1_ref[...], ebg1_ref[...], eWc1_ref[...], ebc1_ref[...])

    @pl.when(t >= _T_IN)
    def _dec():
        get_x = lambda i: cur_ref[slice(_R * i, _R * (i + 1)), :]
        h1 = _two_layers(get_x,
                         dWg0_ref[...], dbg0_ref[...], dWc0_ref[...], dbc0_ref[...],
                         dWg1_ref[...], dbg1_ref[...], dWc1_ref[...], dbc1_ref[...])
        proj = _dot(h1, pW_ref[...]) + pb_ref[...]
        cur_ref[...] = proj
        out_ref[0] = proj.reshape(_B, _N, _D)


def kernel(encoder_inputs, decoder_inputs, supports,
           enc_Wg0, enc_bg0, enc_Wc0, enc_bc0,
           enc_Wg1, enc_bg1, enc_Wc1, enc_bc1,
           dec_Wg0, dec_bg0, dec_Wc0, dec_bc0,
           dec_Wg1, dec_bg1, dec_Wc1, dec_bc1,
           proj_W, proj_b):
    del decoder_inputs

    Abd = jnp.kron(jnp.eye(_BB, dtype=jnp.float32), supports)

    biases = [b.reshape(1, -1) for b in
              (enc_bg0, enc_bc0, enc_bg1, enc_bc1,
               dec_bg0, dec_bc0, dec_bg1, dec_bc1, proj_b)]
    (ebg0, ebc0, ebg1, ebc1, dbg0, dbc0, dbg1, dbc1, pb) = biases

    def _const(shape):
        n = len(shape)
        return pl.BlockSpec(shape, lambda t: (0,) * n)

    out = pl.pallas_call(
        _body,
        grid=(_T_TOT,),
        in_specs=[
            pl.BlockSpec((1, _B, _N, _D),
                         lambda t: (jnp.minimum(t, _T_IN - 1), 0, 0, 0)),
            _const((_R, _R)),
            _const(enc_Wg0.shape), _const(ebg0.shape),
            _const(enc_Wc0.shape), _const(ebc0.shape),
            _const(enc_Wg1.shape), _const(ebg1.shape),
            _const(enc_Wc1.shape), _const(ebc1.shape),
            _const(dec_Wg0.shape), _const(dbg0.shape),
            _const(dec_Wc0.shape), _const(dbc0.shape),
            _const(dec_Wg1.shape), _const(dbg1.shape),
            _const(dec_Wc1.shape), _const(dbc1.shape),
            _const(proj_W.shape), _const(pb.shape),
        ],
        out_specs=pl.BlockSpec((1, _B, _N, _D),
                               lambda t: (jnp.maximum(t - _T_IN, 0), 0, 0, 0)),
        out_shape=jax.ShapeDtypeStruct((_T_OUT, _B, _N, _D), jnp.float32),
        scratch_shapes=[
            pltpu.VMEM((_B * _N, _H), jnp.float32),
            pltpu.VMEM((_B * _N, _H), jnp.float32),
            pltpu.VMEM((_B * _N, _D), jnp.float32),
            pltpu.VMEM((_B * _N, _D + _H), jnp.float32),
            pltpu.VMEM((_B * _N, _D + _H), jnp.float32),
            pltpu.VMEM((_B * _N, _H), jnp.float32),
        ],
        compiler_params=pltpu.CompilerParams(
            dimension_semantics=("arbitrary",)),
    )(encoder_inputs, Abd,
      enc_Wg0, ebg0, enc_Wc0, ebc0,
      enc_Wg1, ebg1, enc_Wc1, ebc1,
      dec_Wg0, dbg0, dec_Wc0, dbc0,
      dec_Wg1, dbg1, dec_Wc1, dbc1,
      proj_W, pb)
    return out.reshape(_T_OUT, _B, _N * _D)


# speedup vs baseline: 1.0869x; 1.0869x over previous
import jax
import jax.numpy as jnp
from jax.experimental import pallas as pl
from jax.experimental.pallas import tpu as pltpu

_B, _T_IN, _T_OUT, _N, _D, _H = 64, 32, 32, 64, 128, 64
_T_TOT = _T_IN + _T_OUT
_BB = 2
_R = _BB * _N
_NCHUNK = _B // _BB
_NBLK = 1
_RB = (_B * _N) // _NBLK


def _dot(a, b):
    return jax.lax.dot(a, b, preferred_element_type=jnp.float32)


def _body(x_ref, Abd_ref,
          eWg0_ref, ebg0_ref, eWc0_ref, ebc0_ref,
          eWg1_ref, ebg1_ref, eWc1_ref, ebc1_ref,
          dWg0_ref, dbg0_ref, dWc0_ref, dbc0_ref,
          dWg1_ref, dbg1_ref, dWc1_ref, dbc1_ref,
          pW_ref, pb_ref,
          out_ref, h0_ref, h1_ref, cur_ref, convg_ref, convc_ref, rh_ref,
          u_ref):
    t = pl.program_id(0)
    Abd = Abd_ref[...]

    @pl.when(t == 0)
    def _init():
        h0_ref[...] = jnp.zeros(h0_ref.shape, jnp.float32)
        h1_ref[...] = jnp.zeros(h1_ref.shape, jnp.float32)
        cur_ref[...] = jnp.zeros(cur_ref.shape, jnp.float32)

    def _cell(get_x, Dx, h_ref, Wg, bg, Wc, bc):
        W = Dx + _H
        for i in range(_NCHUNK):
            sl = slice(_R * i, _R * (i + 1))
            xh = jnp.concatenate([get_x(i), h_ref[sl, :]], axis=1)
            convg_ref[sl, :W] = _dot(Abd, xh)
        for j in range(_NBLK):
            bl = slice(_RB * j, _RB * (j + 1))
            g = jax.nn.sigmoid(_dot(convg_ref[bl, :W], Wg) + bg)
            u_ref[bl, :] = g[:, _H:]
            rh_ref[bl, :] = g[:, :_H] * h_ref[bl, :]
        for i in range(_NCHUNK):
            sl = slice(_R * i, _R * (i + 1))
            xrh = jnp.concatenate([get_x(i), rh_ref[sl, :]], axis=1)
            convc_ref[sl, :W] = _dot(Abd, xrh)
        for j in range(_NBLK):
            bl = slice(_RB * j, _RB * (j + 1))
            c = jnp.tanh(_dot(convc_ref[bl, :W], Wc) + bc)
            u = u_ref[bl, :]
            h_ref[bl, :] = u * h_ref[bl, :] + (1.0 - u) * c

    def _two_layers(get_x0, Wg0, bg0, Wc0, bc0, Wg1, bg1, Wc1, bc1):
        _cell(get_x0, _D, h0_ref, Wg0, bg0, Wc0, bc0)
        get_h0 = lambda i: h0_ref[slice(_R * i, _R * (i + 1)), :]
        _cell(get_h0, _H, h1_ref, Wg1, bg1, Wc1, bc1)

    @pl.when(t < _T_IN)
    def _enc():
        get_x = lambda i: x_ref[0, _BB * i:_BB * (i + 1)].reshape(_R, _D)
        _two_layers(get_x,
                    eWg0_ref[...], ebg0_ref[...], eWc0_ref[...], ebc0_ref[...],
                    eWg1_ref[...], ebg1_ref[...], eWc1_ref[...], ebc1_ref[...])

    @pl.when(t >= _T_IN)
    def _dec():
        get_x = lambda i: cur_ref[slice(_R * i, _R * (i + 1)), :]
        _two_layers(get_x,
                    dWg0_ref[...], dbg0_ref[...], dWc0_ref[...], dbc0_ref[...],
                    dWg1_ref[...], dbg1_ref[...], dWc1_ref[...], dbc1_ref[...])
        for j in range(_NBLK):
            bl = slice(_RB * j, _RB * (j + 1))
            p = _dot(h1_ref[bl, :], pW_ref[...]) + pb_ref[...]
            cur_ref[bl, :] = p
            out_ref[0, (_RB // _N) * j:(_RB // _N) * (j + 1)] = (
                p.reshape(_RB // _N, _N, _D))


def kernel(encoder_inputs, decoder_inputs, supports,
           enc_Wg0, enc_bg0, enc_Wc0, enc_bc0,
           enc_Wg1, enc_bg1, enc_Wc1, enc_bc1,
           dec_Wg0, dec_bg0, dec_Wc0, dec_bc0,
           dec_Wg1, dec_bg1, dec_Wc1, dec_bc1,
           proj_W, proj_b):
    del decoder_inputs

    Abd = jnp.kron(jnp.eye(_BB, dtype=jnp.float32), supports)

    biases = [b.reshape(1, -1) for b in
              (enc_bg0, enc_bc0, enc_bg1, enc_bc1,
               dec_bg0, dec_bc0, dec_bg1, dec_bc1, proj_b)]
    (ebg0, ebc0, ebg1, ebc1, dbg0, dbc0, dbg1, dbc1, pb) = biases

    def _const(shape):
        n = len(shape)
        return pl.BlockSpec(shape, lambda t: (0,) * n)

    out = pl.pallas_call(
        _body,
        grid=(_T_TOT,),
        in_specs=[
            pl.BlockSpec((1, _B, _N, _D),
                         lambda t: (jnp.minimum(t, _T_IN - 1), 0, 0, 0)),
            _const((_R, _R)),
            _const(enc_Wg0.shape), _const(ebg0.shape),
            _const(enc_Wc0.shape), _const(ebc0.shape),
            _const(enc_Wg1.shape), _const(ebg1.shape),
            _const(enc_Wc1.shape), _const(ebc1.shape),
            _const(dec_Wg0.shape), _const(dbg0.shape),
            _const(dec_Wc0.shape), _const(dbc0.shape),
            _const(dec_Wg1.shape), _const(dbg1.shape),
            _const(dec_Wc1.shape), _const(dbc1.shape),
            _const(proj_W.shape), _const(pb.shape),
        ],
        out_specs=pl.BlockSpec((1, _B, _N, _D),
                               lambda t: (jnp.maximum(t - _T_IN, 0), 0, 0, 0)),
        out_shape=jax.ShapeDtypeStruct((_T_OUT, _B, _N, _D), jnp.float32),
        scratch_shapes=[
            pltpu.VMEM((_B * _N, _H), jnp.float32),
            pltpu.VMEM((_B * _N, _H), jnp.float32),
            pltpu.VMEM((_B * _N, _D), jnp.float32),
            pltpu.VMEM((_B * _N, _D + _H), jnp.float32),
            pltpu.VMEM((_B * _N, _D + _H), jnp.float32),
            pltpu.VMEM((_B * _N, _H), jnp.float32),
            pltpu.VMEM((_B * _N, _H), jnp.float32),
        ],
        compiler_params=pltpu.CompilerParams(
            dimension_semantics=("arbitrary",)),
    )(encoder_inputs, Abd,
      enc_Wg0, ebg0, enc_Wc0, ebc0,
      enc_Wg1, ebg1, enc_Wc1, ebc1,
      dec_Wg0, dbg0, dec_Wc0, dbc0,
      dec_Wg1, dbg1, dec_Wc1, dbc1,
      proj_W, pb)
    return out.reshape(_T_OUT, _B, _N * _D)


# speedup vs baseline: 1.0887x; 1.0017x over previous
import jax
import jax.numpy as jnp
from jax.experimental import pallas as pl
from jax.experimental.pallas import tpu as pltpu

_B, _T_IN, _T_OUT, _N, _D, _H = 64, 32, 32, 64, 128, 64
_T_TOT = _T_IN + _T_OUT
_BB = 2
_R = _BB * _N
_NCHUNK = _B // _BB
_NBLK = 1
_RB = (_B * _N) // _NBLK


def _dot(a, b):
    return jax.lax.dot(a, b, preferred_element_type=jnp.float32)


def _body(x_ref, Abd_ref,
          eWg0_ref, ebg0_ref, eWc0_ref, ebc0_ref,
          eWg1_ref, ebg1_ref, eWc1_ref, ebc1_ref,
          dWg0_ref, dbg0_ref, dWc0_ref, dbc0_ref,
          dWg1_ref, dbg1_ref, dWc1_ref, dbc1_ref,
          pW_ref, pb_ref,
          out_ref, h0_ref, h1_ref, cur_ref, convg_ref, convc_ref, rh_ref,
          u_ref):
    t = pl.program_id(0)
    Abd = Abd_ref[...]

    @pl.when(t == 0)
    def _init():
        h0_ref[...] = jnp.zeros(h0_ref.shape, jnp.float32)
        h1_ref[...] = jnp.zeros(h1_ref.shape, jnp.float32)
        cur_ref[...] = jnp.zeros(cur_ref.shape, jnp.float32)

    def _cell(get_x, Dx, h_ref, Wg, bg, Wc, bc):
        W = Dx + _H
        for i in range(_NCHUNK):
            sl = slice(_R * i, _R * (i + 1))
            xh = jnp.concatenate([get_x(i), h_ref[sl, :]], axis=1)
            convg_ref[sl, :W] = _dot(Abd, xh)
        for j in range(_NBLK):
            bl = slice(_RB * j, _RB * (j + 1))
            g = jax.nn.sigmoid(_dot(convg_ref[bl, :W], Wg) + bg)
            u_ref[bl, :] = g[:, _H:]
            rh_ref[bl, :] = g[:, :_H] * h_ref[bl, :]
        for i in range(_NCHUNK):
            sl = slice(_R * i, _R * (i + 1))
            xrh = jnp.concatenate([get_x(i), rh_ref[sl, :]], axis=1)
            convc_ref[sl, :W] = _dot(Abd, xrh)
        for j in range(_NBLK):
            bl = slice(_RB * j, _RB * (j + 1))
            c = jnp.tanh(_dot(convc_ref[bl, :W], Wc) + bc)
            u = u_ref[bl, :]
            h_ref[bl, :] = u * h_ref[bl, :] + (1.0 - u) * c

    def _two_layers(get_x0, Wg0, bg0, Wc0, bc0, Wg1, bg1, Wc1, bc1):
        _cell(get_x0, _D, h0_ref, Wg0, bg0, Wc0, bc0)
        get_h0 = lambda i: h0_ref[slice(_R * i, _R * (i + 1)), :]
        _cell(get_h0, _H, h1_ref, Wg1, bg1, Wc1, bc1)

    @pl.when(t < _T_IN)
    def _enc():
        get_x = lambda i: x_ref[0, _R * i:_R * (i + 1), :]
        _two_layers(get_x,
                    eWg0_ref[...], ebg0_ref[...], eWc0_ref[...], ebc0_ref[...],
                    eWg1_ref[...], ebg1_ref[...], eWc1_ref[...], ebc1_ref[...])

    @pl.when(t >= _T_IN)
    def _dec():
        get_x = lambda i: cur_ref[slice(_R * i, _R * (i + 1)), :]
        _two_layers(get_x,
                    dWg0_ref[...], dbg0_ref[...], dWc0_ref[...], dbc0_ref[...],
                    dWg1_ref[...], dbg1_ref[...], dWc1_ref[...], dbc1_ref[...])
        for j in range(_NBLK):
            bl = slice(_RB * j, _RB * (j + 1))
            p = _dot(h1_ref[bl, :], pW_ref[...]) + pb_ref[...]
            cur_ref[bl, :] = p
            out_ref[0, bl] = p


def kernel(encoder_inputs, decoder_inputs, supports,
           enc_Wg0, enc_bg0, enc_Wc0, enc_bc0,
           enc_Wg1, enc_bg1, enc_Wc1, enc_bc1,
           dec_Wg0, dec_bg0, dec_Wc0, dec_bc0,
           dec_Wg1, dec_bg1, dec_Wc1, dec_bc1,
           proj_W, proj_b):
    del decoder_inputs

    Abd = jnp.kron(jnp.eye(_BB, dtype=jnp.float32), supports)

    biases = [b.reshape(1, -1) for b in
              (enc_bg0, enc_bc0, enc_bg1, enc_bc1,
               dec_bg0, dec_bc0, dec_bg1, dec_bc1, proj_b)]
    (ebg0, ebc0, ebg1, ebc1, dbg0, dbc0, dbg1, dbc1, pb) = biases

    def _const(shape):
        n = len(shape)
        return pl.BlockSpec(shape, lambda t: (0,) * n)

    out = pl.pallas_call(
        _body,
        grid=(_T_TOT,),
        in_specs=[
            pl.BlockSpec((1, _B * _N, _D),
                         lambda t: (jnp.minimum(t, _T_IN - 1), 0, 0)),
            _const((_R, _R)),
            _const(enc_Wg0.shape), _const(ebg0.shape),
            _const(enc_Wc0.shape), _const(ebc0.shape),
            _const(enc_Wg1.shape), _const(ebg1.shape),
            _const(enc_Wc1.shape), _const(ebc1.shape),
            _const(dec_Wg0.shape), _const(dbg0.shape),
            _const(dec_Wc0.shape), _const(dbc0.shape),
            _const(dec_Wg1.shape), _const(dbg1.shape),
            _const(dec_Wc1.shape), _const(dbc1.shape),
            _const(proj_W.shape), _const(pb.shape),
        ],
        out_specs=pl.BlockSpec((1, _B * _N, _D),
                               lambda t: (jnp.maximum(t - _T_IN, 0), 0, 0)),
        out_shape=jax.ShapeDtypeStruct((_T_OUT, _B * _N, _D), jnp.float32),
        scratch_shapes=[
            pltpu.VMEM((_B * _N, _H), jnp.float32),
            pltpu.VMEM((_B * _N, _H), jnp.float32),
            pltpu.VMEM((_B * _N, _D), jnp.float32),
            pltpu.VMEM((_B * _N, _D + _H), jnp.float32),
            pltpu.VMEM((_B * _N, _D + _H), jnp.float32),
            pltpu.VMEM((_B * _N, _H), jnp.float32),
            pltpu.VMEM((_B * _N, _H), jnp.float32),
        ],
        compiler_params=pltpu.CompilerParams(
            dimension_semantics=("arbitrary",)),
    )(encoder_inputs.reshape(_T_IN, _B * _N, _D), Abd,
      enc_Wg0, ebg0, enc_Wc0, ebc0,
      enc_Wg1, ebg1, enc_Wc1, ebc1,
      dec_Wg0, dbg0, dec_Wc0, dbc0,
      dec_Wg1, dbg1, dec_Wc1, dbc1,
      proj_W, pb)
    return out.reshape(_T_OUT, _B, _N * _D)


# speedup vs baseline: 1.1111x; 1.0205x over previous
import jax
import jax.numpy as jnp
from jax.experimental import pallas as pl
from jax.experimental.pallas import tpu as pltpu

_B, _T_IN, _T_OUT, _N, _D, _H = 64, 32, 32, 64, 128, 64
_T_TOT = _T_IN + _T_OUT
_BB = 4
_R = _BB * _N
_NCHUNK = _B // _BB
_NBLK = 1
_RB = (_B * _N) // _NBLK


def _dot(a, b):
    return jax.lax.dot(a, b, preferred_element_type=jnp.float32)


def _body(x_ref, Abd_ref,
          eWg0_ref, ebg0_ref, eWc0_ref, ebc0_ref,
          eWg1_ref, ebg1_ref, eWc1_ref, ebc1_ref,
          dWg0_ref, dbg0_ref, dWc0_ref, dbc0_ref,
          dWg1_ref, dbg1_ref, dWc1_ref, dbc1_ref,
          pW_ref, pb_ref,
          out_ref, h0_ref, h1_ref, cur_ref, convg_ref, convc_ref, rh_ref,
          u_ref):
    t = pl.program_id(0)
    Abd = Abd_ref[...]

    @pl.when(t == 0)
    def _init():
        h0_ref[...] = jnp.zeros(h0_ref.shape, jnp.float32)
        h1_ref[...] = jnp.zeros(h1_ref.shape, jnp.float32)
        cur_ref[...] = jnp.zeros(cur_ref.shape, jnp.float32)

    def _cell(get_x, Dx, h_ref, Wg, bg, Wc, bc):
        W = Dx + _H
        for i in range(_NCHUNK):
            sl = slice(_R * i, _R * (i + 1))
            xh = jnp.concatenate([get_x(i), h_ref[sl, :]], axis=1)
            convg_ref[sl, :W] = _dot(Abd, xh)
        for j in range(_NBLK):
            bl = slice(_RB * j, _RB * (j + 1))
            g = jax.nn.sigmoid(_dot(convg_ref[bl, :W], Wg) + bg)
            u_ref[bl, :] = g[:, _H:]
            rh_ref[bl, :] = g[:, :_H] * h_ref[bl, :]
        for i in range(_NCHUNK):
            sl = slice(_R * i, _R * (i + 1))
            xrh = jnp.concatenate([get_x(i), rh_ref[sl, :]], axis=1)
            convc_ref[sl, :W] = _dot(Abd, xrh)
        for j in range(_NBLK):
            bl = slice(_RB * j, _RB * (j + 1))
            c = jnp.tanh(_dot(convc_ref[bl, :W], Wc) + bc)
            u = u_ref[bl, :]
            h_ref[bl, :] = u * h_ref[bl, :] + (1.0 - u) * c

    def _two_layers(get_x0, Wg0, bg0, Wc0, bc0, Wg1, bg1, Wc1, bc1):
        _cell(get_x0, _D, h0_ref, Wg0, bg0, Wc0, bc0)
        get_h0 = lambda i: h0_ref[slice(_R * i, _R * (i + 1)), :]
        _cell(get_h0, _H, h1_ref, Wg1, bg1, Wc1, bc1)

    @pl.when(t < _T_IN)
    def _enc():
        get_x = lambda i: x_ref[0, _R * i:_R * (i + 1), :]
        _two_layers(get_x,
                    eWg0_ref[...], ebg0_ref[...], eWc0_ref[...], ebc0_ref[...],
                    eWg1_ref[...], ebg1_ref[...], eWc1_ref[...], ebc1_ref[...])

    @pl.when(t >= _T_IN)
    def _dec():
        get_x = lambda i: cur_ref[slice(_R * i, _R * (i + 1)), :]
        _two_layers(get_x,
                    dWg0_ref[...], dbg0_ref[...], dWc0_ref[...], dbc0_ref[...],
                    dWg1_ref[...], dbg1_ref[...], dWc1_ref[...], dbc1_ref[...])
        for j in range(_NBLK):
            bl = slice(_RB * j, _RB * (j + 1))
            p = _dot(h1_ref[bl, :], pW_ref[...]) + pb_ref[...]
            cur_ref[bl, :] = p
            out_ref[0, bl] = p


def kernel(encoder_inputs, decoder_inputs, supports,
           enc_Wg0, enc_bg0, enc_Wc0, enc_bc0,
           enc_Wg1, enc_bg1, enc_Wc1, enc_bc1,
           dec_Wg0, dec_bg0, dec_Wc0, dec_bc0,
           dec_Wg1, dec_bg1, dec_Wc1, dec_bc1,
           proj_W, proj_b):
    del decoder_inputs

    Abd = jnp.kron(jnp.eye(_BB, dtype=jnp.float32), supports)

    biases = [b.reshape(1, -1) for b in
              (enc_bg0, enc_bc0, enc_bg1, enc_bc1,
               dec_bg0, dec_bc0, dec_bg1, dec_bc1, proj_b)]
    (ebg0, ebc0, ebg1, ebc1, dbg0, dbc0, dbg1, dbc1, pb) = biases

    def _const(shape):
        n = len(shape)
        return pl.BlockSpec(shape, lambda t: (0,) * n)

    out = pl.pallas_call(
        _body,
        grid=(_T_TOT,),
        in_specs=[
            pl.BlockSpec((1, _B * _N, _D),
                         lambda t: (jnp.minimum(t, _T_IN - 1), 0, 0)),
            _const((_R, _R)),
            _const(enc_Wg0.shape), _const(ebg0.shape),
            _const(enc_Wc0.shape), _const(ebc0.shape),
            _const(enc_Wg1.shape), _const(ebg1.shape),
            _const(enc_Wc1.shape), _const(ebc1.shape),
            _const(dec_Wg0.shape), _const(dbg0.shape),
            _const(dec_Wc0.shape), _const(dbc0.shape),
            _const(dec_Wg1.shape), _const(dbg1.shape),
            _const(dec_Wc1.shape), _const(dbc1.shape),
            _const(proj_W.shape), _const(pb.shape),
        ],
        out_specs=pl.BlockSpec((1, _B * _N, _D),
                               lambda t: (jnp.maximum(t - _T_IN, 0), 0, 0)),
        out_shape=jax.ShapeDtypeStruct((_T_OUT, _B * _N, _D), jnp.float32),
        scratch_shapes=[
            pltpu.VMEM((_B * _N, _H), jnp.float32),
            pltpu.VMEM((_B * _N, _H), jnp.float32),
            pltpu.VMEM((_B * _N, _D), jnp.float32),
            pltpu.VMEM((_B * _N, _D + _H), jnp.float32),
            pltpu.VMEM((_B * _N, _D + _H), jnp.float32),
            pltpu.VMEM((_B * _N, _H), jnp.float32),
            pltpu.VMEM((_B * _N, _H), jnp.float32),
        ],
        compiler_params=pltpu.CompilerParams(
            dimension_semantics=("arbitrary",)),
    )(encoder_inputs.reshape(_T_IN, _B * _N, _D), Abd,
      enc_Wg0, ebg0, enc_Wc0, ebc0,
      enc_Wg1, ebg1, enc_Wc1, ebc1,
      dec_Wg0, dbg0, dec_Wc0, dbc0,
      dec_Wg1, dbg1, dec_Wc1, dbc1,
      proj_W, pb)
    return out.reshape(_T_OUT, _B, _N * _D)
